# baseline (device time: 17507 ns/iter reference)
import os

import jax
import jax.numpy as jnp
from jax import lax
from jax.experimental import pallas as pl
from jax.experimental.pallas import tpu as pltpu

N_DEV = 4

_NO_COMM = os.environ.get("KERNEL_NO_COMM") == "1"
_TRIVIAL = os.environ.get("KERNEL_TRIVIAL") == "1"


def kernel(x, Wq, Wo, K_ext, V_ext):
    B, Sq, D = x.shape
    _, Skv, Hkv, Dh = K_ext.shape
    Dq = Wq.shape[1]
    Hq_loc = Dq // Dh
    G = Hq_loc // 4
    R = B * Sq
    D2 = D // 2

    bf16 = jnp.bfloat16
    Wqb = (Wq * 0.125).astype(bf16)
    Wob = Wo.astype(bf16)

    idx = lax.axis_index("i")
    K_loc, V_loc = lax.switch(
        idx,
        [
            (lambda i=i: (
                lax.slice_in_dim(K_ext, i * G, (i + 1) * G, axis=2),
                lax.slice_in_dim(V_ext, i * G, (i + 1) * G, axis=2),
            ))
            for i in range(N_DEV)
        ],
    )
    K_loc = K_loc.astype(bf16).reshape(B * Skv, G * Dh)
    V_loc = V_loc.astype(bf16).reshape(B * Skv, G * Dh)

    def body(x_ref, wq_ref, wo_ref, k_ref, v_ref, out_ref,
             pbuf, rbuf, send_sems, recv_sems):
        my = lax.axis_index("i")
        p1 = my + 1 - 2 * lax.rem(my, 2)
        p2 = (N_DEV - 1) - my

        if not _NO_COMM:
            barrier_sem = pltpu.get_barrier_semaphore()
            for nbr in (p1, p2):
                pl.semaphore_signal(
                    barrier_sem, inc=1,
                    device_id=(nbr,), device_id_type=pl.DeviceIdType.MESH,
                )

        if _TRIVIAL:
            for b in range(B):
                out_ref[b] = x_ref[b].astype(bf16)
            return

        wq = wq_ref[:]
        wo = wo_ref[:]
        kb = k_ref[:]
        vb = v_ref[:]

        def ph1(c):
            r = pltpu.make_async_remote_copy(
                src_ref=pbuf.at[c], dst_ref=rbuf.at[c],
                send_sem=send_sems.at[c], recv_sem=recv_sems.at[c],
                device_id=((p1, p2)[c % 2],),
                device_id_type=pl.DeviceIdType.MESH,
            )
            r.start()
            return r

        def ph2(c):
            r = pltpu.make_async_remote_copy(
                src_ref=pbuf.at[c], dst_ref=rbuf.at[4 + c],
                send_sem=send_sems.at[4 + c], recv_sem=recv_sems.at[4 + c],
                device_id=((p2, p1)[c % 2],),
                device_id_type=pl.DeviceIdType.MESH,
            )
            r.start()
            return r

        chunks = [None] * 4
        r1 = [None] * 4
        for b in range(B):
            rows = slice(b * Sq, (b + 1) * Sq)
            q = lax.dot_general(x_ref[b].astype(bf16), wq,
                                (((1,), (0,)), ((), ())),
                                preferred_element_type=jnp.float32
                                ).astype(bf16)

            attn_cols = []
            for g in range(G):
                qs = jnp.concatenate(
                    [q[:, (g * 4 + hh) * Dh:(g * 4 + hh + 1) * Dh]
                     for hh in range(4)], axis=0)
                kg = kb[rows, g * Dh:(g + 1) * Dh]
                vg = vb[rows, g * Dh:(g + 1) * Dh]
                s = lax.dot_general(qs, kg, (((1,), (1,)), ((), ())),
                                    preferred_element_type=jnp.float32)
                p = jnp.exp(s)
                l = jnp.sum(p, axis=1, keepdims=True)
                o = lax.dot_general(p.astype(bf16), vg,
                                    (((1,), (0,)), ((), ())),
                                    preferred_element_type=jnp.float32)
                on = (o / l).astype(bf16)
                attn_cols.extend(
                    on[hh * Sq:(hh + 1) * Sq, :] for hh in range(4))
            attn_b = jnp.concatenate(attn_cols, axis=1)

            if _NO_COMM:
                pb = lax.dot_general(attn_b, wo, (((1,), (0,)), ((), ())),
                                     preferred_element_type=jnp.float32)
                out_ref[b] = pb.astype(bf16)
                continue

            if b == 0:
                pl.semaphore_wait(barrier_sem, 2)

            for half in range(2):
                c = 2 * b + half
                pc = lax.dot_general(
                    attn_b, wo[:, half * D2:(half + 1) * D2],
                    (((1,), (0,)), ((), ())),
                    preferred_element_type=jnp.float32)
                chunks[c] = pc
                pbuf[c] = pc.astype(bf16)
                r1[c] = ph1(c)

        if _NO_COMM:
            return

        sums = [None] * 4
        r2 = [None] * 4
        for c in range(4):
            r1[c].wait_recv()
            sums[c] = chunks[c] + rbuf[c].astype(jnp.float32)
            r1[c].wait_send()
            pbuf[c] = sums[c].astype(bf16)
            r2[c] = ph2(c)
        for c in range(4):
            b, half = divmod(c, 2)
            r2[c].wait_recv()
            out_ref[b, :, half * D2:(half + 1) * D2] = (
                sums[c] + rbuf[4 + c].astype(jnp.float32)).astype(bf16)
        for c in range(4):
            r2[c].wait_send()

    out2 = pl.pallas_call(
        body,
        out_shape=jax.ShapeDtypeStruct((B, Sq, D), jnp.bfloat16),
        in_specs=[pl.BlockSpec(memory_space=pltpu.VMEM)] * 5,
        out_specs=pl.BlockSpec(memory_space=pltpu.VMEM),
        scratch_shapes=[
            pltpu.VMEM((4, Sq, D2), jnp.bfloat16),
            pltpu.VMEM((8, Sq, D2), jnp.bfloat16),
            pltpu.SemaphoreType.DMA((8,)),
            pltpu.SemaphoreType.DMA((8,)),
        ],
        **({} if _NO_COMM
           else dict(compiler_params=pltpu.CompilerParams(collective_id=0))),
    )(x, Wqb, Wob, K_loc, V_loc)
    return out2


# device time: 13755 ns/iter; 1.2728x vs baseline; 1.2728x over previous
import os

import jax
import jax.numpy as jnp
from jax import lax
from jax.experimental import pallas as pl
from jax.experimental.pallas import tpu as pltpu

N_DEV = 4

_NO_COMM = os.environ.get("KERNEL_NO_COMM") == "1"
_TRIVIAL = os.environ.get("KERNEL_TRIVIAL") == "1"


def kernel(x, Wq, Wo, K_ext, V_ext):
    B, Sq, D = x.shape
    _, Skv, Hkv, Dh = K_ext.shape
    Dq = Wq.shape[1]
    Hq_loc = Dq // Dh
    G = Hq_loc // 4
    R = B * Sq
    D2 = D // 2

    bf16 = jnp.bfloat16
    Wqb = (Wq * 0.125).astype(bf16)
    Wob = Wo.astype(bf16)

    idx = lax.axis_index("i")
    K_loc = lax.dynamic_slice_in_dim(K_ext, idx * G, G, axis=2)
    V_loc = lax.dynamic_slice_in_dim(V_ext, idx * G, G, axis=2)
    K_loc = K_loc.astype(bf16).reshape(B * Skv, G * Dh)
    V_loc = V_loc.astype(bf16).reshape(B * Skv, G * Dh)

    def body(x_ref, wq_ref, wo_ref, k_ref, v_ref, out_ref,
             pbuf, rbuf, send_sems, recv_sems):
        my = lax.axis_index("i")
        p1 = my + 1 - 2 * lax.rem(my, 2)
        p2 = (N_DEV - 1) - my

        if not _NO_COMM:
            barrier_sem = pltpu.get_barrier_semaphore()
            for nbr in (p1, p2):
                pl.semaphore_signal(
                    barrier_sem, inc=1,
                    device_id=(nbr,), device_id_type=pl.DeviceIdType.MESH,
                )

        if _TRIVIAL:
            for b in range(B):
                out_ref[b] = x_ref[b].astype(bf16)
            return

        wq = wq_ref[:]
        wo = wo_ref[:]
        kb = k_ref[:]
        vb = v_ref[:]

        def ph1(c):
            r = pltpu.make_async_remote_copy(
                src_ref=pbuf.at[c], dst_ref=rbuf.at[c],
                send_sem=send_sems.at[c], recv_sem=recv_sems.at[c],
                device_id=((p1, p2)[c % 2],),
                device_id_type=pl.DeviceIdType.MESH,
            )
            r.start()
            return r

        def ph2(c):
            r = pltpu.make_async_remote_copy(
                src_ref=pbuf.at[c], dst_ref=rbuf.at[4 + c],
                send_sem=send_sems.at[4 + c], recv_sem=recv_sems.at[4 + c],
                device_id=((p2, p1)[c % 2],),
                device_id_type=pl.DeviceIdType.MESH,
            )
            r.start()
            return r

        chunks = [None] * 4
        r1 = [None] * 4
        for b in range(B):
            rows = slice(b * Sq, (b + 1) * Sq)
            q = lax.dot_general(x_ref[b].astype(bf16), wq,
                                (((1,), (0,)), ((), ())),
                                preferred_element_type=jnp.float32
                                ).astype(bf16)

            attn_cols = []
            for g in range(G):
                qs = jnp.concatenate(
                    [q[:, (g * 4 + hh) * Dh:(g * 4 + hh + 1) * Dh]
                     for hh in range(4)], axis=0)
                kg = kb[rows, g * Dh:(g + 1) * Dh]
                vg = vb[rows, g * Dh:(g + 1) * Dh]
                s = lax.dot_general(qs, kg, (((1,), (1,)), ((), ())),
                                    preferred_element_type=jnp.float32)
                p = jnp.exp(s)
                l = jnp.sum(p, axis=1, keepdims=True)
                o = lax.dot_general(p.astype(bf16), vg,
                                    (((1,), (0,)), ((), ())),
                                    preferred_element_type=jnp.float32)
                on = (o / l).astype(bf16)
                attn_cols.extend(
                    on[hh * Sq:(hh + 1) * Sq, :] for hh in range(4))
            attn_b = jnp.concatenate(attn_cols, axis=1)

            if _NO_COMM:
                pb = lax.dot_general(attn_b, wo, (((1,), (0,)), ((), ())),
                                     preferred_element_type=jnp.float32)
                out_ref[b] = pb.astype(bf16)
                continue

            if b == 0:
                pl.semaphore_wait(barrier_sem, 2)

            for half in range(2):
                c = 2 * b + half
                pc = lax.dot_general(
                    attn_b, wo[:, half * D2:(half + 1) * D2],
                    (((1,), (0,)), ((), ())),
                    preferred_element_type=jnp.float32)
                chunks[c] = pc
                pbuf[c] = pc.astype(bf16)
                r1[c] = ph1(c)

        if _NO_COMM:
            return

        sums = [None] * 4
        r2 = [None] * 4
        for c in range(4):
            r1[c].wait_recv()
            sums[c] = chunks[c] + rbuf[c].astype(jnp.float32)
            r1[c].wait_send()
            pbuf[c] = sums[c].astype(bf16)
            r2[c] = ph2(c)
        for c in range(4):
            b, half = divmod(c, 2)
            r2[c].wait_recv()
            out_ref[b, :, half * D2:(half + 1) * D2] = (
                sums[c] + rbuf[4 + c].astype(jnp.float32)).astype(bf16)
        for c in range(4):
            r2[c].wait_send()

    out2 = pl.pallas_call(
        body,
        out_shape=jax.ShapeDtypeStruct((B, Sq, D), jnp.bfloat16),
        in_specs=[pl.BlockSpec(memory_space=pltpu.VMEM)] * 5,
        out_specs=pl.BlockSpec(memory_space=pltpu.VMEM),
        scratch_shapes=[
            pltpu.VMEM((4, Sq, D2), jnp.bfloat16),
            pltpu.VMEM((8, Sq, D2), jnp.bfloat16),
            pltpu.SemaphoreType.DMA((8,)),
            pltpu.SemaphoreType.DMA((8,)),
        ],
        **({} if _NO_COMM
           else dict(compiler_params=pltpu.CompilerParams(collective_id=0))),
    )(x, Wqb, Wob, K_loc, V_loc)
    return out2
